# Initial kernel scaffold; baseline (speedup 1.0000x reference)
import jax
import jax.numpy as jnp
from jax.experimental import pallas as pl


def kernel(x, edge_index, W_l, W_r, b):
    raise NotImplementedError("write your pallas kernel here")


# speedup vs baseline: 6.1292x; 6.1292x over previous
import functools

import jax
import jax.numpy as jnp
from jax import lax
from jax.experimental import pallas as pl
from jax.experimental.pallas import tpu as pltpu
from jax.experimental.pallas import tpu_sc as plsc

N_NODES = 10000
N_EDGES = 320000
D = 128
L = 16

NC = 2
NS = 16
NW = NC * NS

NPAD = 10240
EDGES_PER_TILE = N_EDGES // NW
CHUNK = 80
NCHUNKS = EDGES_PER_TILE // CHUNK
ZROWS = NPAD // NS


def _sc_segment_sum(x, src, dst, zeros_feat, zeros_deg):
    mesh = plsc.VectorSubcoreMesh(
        core_axis_name="c", subcore_axis_name="s", num_cores=NC, num_subcores=NS
    )

    @functools.partial(
        pl.kernel,
        mesh=mesh,
        compiler_params=pltpu.CompilerParams(needs_layout_passes=False),
        out_type=[
            jax.ShapeDtypeStruct((NC * NPAD, D), jnp.float32),
            jax.ShapeDtypeStruct((NC * NPAD,), jnp.float32),
        ],
        scratch_types=[
            pltpu.VMEM((CHUNK,), jnp.int32),
            pltpu.VMEM((CHUNK,), jnp.int32),
            pltpu.VMEM((CHUNK, D), jnp.float32),
            pltpu.VMEM((NPAD,), jnp.float32),
            pltpu.VMEM((NS, ZROWS), jnp.float32),
            pltpu.VMEM_SHARED((NPAD, D), jnp.float32),
            pltpu.VMEM_SHARED((NS * NPAD,), jnp.float32),
            pltpu.SemaphoreType.DMA,
        ],
    )
    def seg_kernel(x_hbm, src_hbm, dst_hbm, zf_hbm, zd_hbm,
                   acc_out, deg_out,
                   src_v, dst_v, rows_v, deg_v, red_v, acc_s, deg_all, sem):
        cid = lax.axis_index("c")
        sid = lax.axis_index("s")
        wid = sid * NC + cid
        zbase = sid * ZROWS

        pltpu.sync_copy(zf_hbm.at[pl.ds(zbase, ZROWS)], acc_s.at[pl.ds(zbase, ZROWS)])
        pltpu.sync_copy(zd_hbm, deg_v)
        plsc.subcore_barrier()

        ones = jnp.ones((L,), jnp.float32)

        def body(j, carry):
            base = wid * EDGES_PER_TILE + j * CHUNK
            pltpu.sync_copy(src_hbm.at[pl.ds(base, CHUNK)], src_v)
            pltpu.sync_copy(dst_hbm.at[pl.ds(base, CHUNK)], dst_v)
            pltpu.async_copy(x_hbm.at[src_v], rows_v, sem).wait()
            pltpu.sync_copy(rows_v, acc_s.at[dst_v], add=True)
            for k in range(CHUNK // L):
                idx = dst_v[pl.ds(k * L, L)]
                plsc.addupdate_scatter(deg_v, [idx], ones)
            return carry

        lax.fori_loop(0, NCHUNKS, body, 0)

        pltpu.sync_copy(deg_v, deg_all.at[pl.ds(sid * NPAD, NPAD)])
        plsc.subcore_barrier()
        for t in range(NS):
            pltpu.sync_copy(deg_all.at[pl.ds(t * NPAD + zbase, ZROWS)],
                            red_v.at[t])
        for j in range(ZROWS // L):
            degsum = red_v[0, pl.ds(j * L, L)]
            for t in range(1, NS):
                degsum = degsum + red_v[t, pl.ds(j * L, L)]
            deg_v[pl.ds(j * L, L)] = degsum

        obase = cid * NPAD + zbase
        pltpu.sync_copy(acc_s.at[pl.ds(zbase, ZROWS)],
                        acc_out.at[pl.ds(obase, ZROWS)])
        pltpu.sync_copy(deg_v.at[pl.ds(0, ZROWS)], deg_out.at[pl.ds(obase, ZROWS)])

    return seg_kernel(x, src, dst, zeros_feat, zeros_deg)


ROWS_BLK = 1000


def _tc_combine_kernel(acc_ref, d0_ref, d1_ref, x_ref, wl_ref, wr_ref, b_ref,
                       out_ref):
    deg = jnp.maximum(d0_ref[...] + d1_ref[...], 1.0)
    agg = (acc_ref[0] + acc_ref[1]) / deg
    out = (jnp.dot(agg, wl_ref[...], preferred_element_type=jnp.float32)
           + jnp.dot(x_ref[...], wr_ref[...], preferred_element_type=jnp.float32)
           + b_ref[...])
    out_ref[...] = jnp.maximum(out, 0.0)


def _tc_combine(acc, d0, d1, x, w_l, w_r, b):
    grid = (N_NODES // ROWS_BLK,)
    return pl.pallas_call(
        _tc_combine_kernel,
        grid=grid,
        in_specs=[
            pl.BlockSpec((NC, ROWS_BLK, D), lambda i: (0, i, 0)),
            pl.BlockSpec((ROWS_BLK, 1), lambda i: (i, 0)),
            pl.BlockSpec((ROWS_BLK, 1), lambda i: (i, 0)),
            pl.BlockSpec((ROWS_BLK, D), lambda i: (i, 0)),
            pl.BlockSpec((D, D), lambda i: (0, 0)),
            pl.BlockSpec((D, D), lambda i: (0, 0)),
            pl.BlockSpec((1, D), lambda i: (0, 0)),
        ],
        out_specs=pl.BlockSpec((ROWS_BLK, D), lambda i: (i, 0)),
        out_shape=jax.ShapeDtypeStruct((N_NODES, D), jnp.float32),
    )(acc, d0, d1, x, w_l, w_r, b)


def kernel(x, edge_index, W_l, W_r, b):
    src = edge_index[0]
    dst = edge_index[1]
    zeros_feat = jnp.zeros((NPAD, D), jnp.float32)
    zeros_deg = jnp.zeros((NPAD,), jnp.float32)
    acc, deg = _sc_segment_sum(x, src, dst, zeros_feat, zeros_deg)
    acc = acc.reshape(NC, NPAD, D)
    deg = deg.reshape(NC, NPAD)
    d0 = deg[0].reshape(NPAD, 1)
    d1 = deg[1].reshape(NPAD, 1)
    return _tc_combine(acc, d0, d1, x, W_l, W_r, b.reshape(1, D))
